# Initial kernel scaffold; baseline (speedup 1.0000x reference)
import jax
import jax.numpy as jnp
from jax.experimental import pallas as pl


def kernel(action_type, x, y, action_table, x_table, y_table):
    raise NotImplementedError("write your pallas kernel here")


# speedup vs baseline: 1.6109x; 1.6109x over previous
import functools

import jax
import jax.numpy as jnp
from jax import lax
from jax.experimental import pallas as pl
from jax.experimental.pallas import tpu as pltpu
from jax.experimental.pallas import tpu_sc as plsc

B, L, D = 4096, 200, 128
N = B * L
NC, NS = 2, 16
NW = NC * NS
PER_W = N // NW
C = 256
NCHUNK = PER_W // C


def _sc_body(at_hbm, xi_hbm, yi_hbm, atab_hbm, xtab_hbm, ytab_hbm, out_hbm,
             ai_v, xi_v, yi_v, arows, xrows, yrows, sem):
    wid = lax.axis_index("s") * NC + lax.axis_index("c")
    base = wid * PER_W

    def chunk(ci, carry):
        off = base + ci * C
        pltpu.sync_copy(at_hbm.at[pl.ds(off, C)], ai_v)
        pltpu.sync_copy(xi_hbm.at[pl.ds(off, C)], xi_v)
        pltpu.sync_copy(yi_hbm.at[pl.ds(off, C)], yi_v)
        pltpu.async_copy(atab_hbm.at[ai_v], arows, sem).wait()
        pltpu.async_copy(xtab_hbm.at[xi_v], xrows, sem).wait()
        pltpu.async_copy(ytab_hbm.at[yi_v], yrows, sem).wait()

        def row(i, c2):
            for j in range(D // 16):
                sl = pl.ds(j * 16, 16)
                arows[i, sl] = arows[i, sl] + xrows[i, sl] + yrows[i, sl]
            return c2

        lax.fori_loop(0, C, row, 0, unroll=False)
        pltpu.sync_copy(arows, out_hbm.at[pl.ds(off, C)])
        return carry

    lax.fori_loop(0, NCHUNK, chunk, 0, unroll=False)


def kernel(action_type, x, y, action_table, x_table, y_table):
    at = action_type.reshape(N).astype(jnp.int32)
    xi = x.reshape(N).astype(jnp.int32)
    yi = y.reshape(N).astype(jnp.int32)

    mesh = plsc.VectorSubcoreMesh(core_axis_name="c", subcore_axis_name="s")
    run = functools.partial(
        pl.kernel,
        mesh=mesh,
        out_type=jax.ShapeDtypeStruct((N, D), jnp.float32),
        scratch_types=[
            pltpu.VMEM((C,), jnp.int32),
            pltpu.VMEM((C,), jnp.int32),
            pltpu.VMEM((C,), jnp.int32),
            pltpu.VMEM((C, D), jnp.float32),
            pltpu.VMEM((C, D), jnp.float32),
            pltpu.VMEM((C, D), jnp.float32),
            pltpu.SemaphoreType.DMA,
        ],
    )(_sc_body)
    out = run(at, xi, yi, action_table, x_table, y_table)
    return out.reshape(B, L, D)
